# baseline (speedup 1.0000x reference)
import functools

import jax
import jax.numpy as jnp
from jax import lax
from jax.experimental import pallas as pl
from jax.experimental.pallas import tpu as pltpu
from jax.experimental.pallas import tpu_sc as plsc

_B = 16384
_D = 32
_G = 8
_NC = 2
_NS = 16
_NW = _NC * _NS
_BPW = _B // _NW
_CHUNK = 16
_NCHUNK = _BPW // _CHUNK


_HALF = _B // 2
_SCH = 512
_NSCH = _HALF // _SCH


def _sc_gather2(user_emb, item_emb, user_idx, item_idx):
    mesh = plsc.ScalarSubcoreMesh(axis_name="core", num_cores=_NC)
    rows = jax.ShapeDtypeStruct((_B, _D), jnp.float32)

    @functools.partial(
        pl.kernel,
        mesh=mesh,
        out_type=(rows, rows),
        scratch_types=[
            pltpu.SMEM((_SCH,), jnp.int32),
            pltpu.SMEM((_SCH,), jnp.int32),
            pltpu.SemaphoreType.DMA,
            pltpu.SemaphoreType.DMA,
            pltpu.SemaphoreType.DMA,
        ],
    )
    def k(uemb_hbm, iemb_hbm, uidx_hbm, iidx_hbm, ue_out, ie_out,
          uidx_s, iidx_s, sem_u, sem_i, sem_x):
        cid = lax.axis_index("core")
        base = cid * _HALF

        @pl.loop(0, _NSCH)
        def _(ch):
            cb = base + ch * _SCH
            pltpu.async_copy(uidx_hbm.at[pl.ds(cb, _SCH)], uidx_s, sem_x).wait()
            pltpu.async_copy(iidx_hbm.at[pl.ds(cb, _SCH)], iidx_s, sem_x).wait()

            @pl.loop(0, _SCH, step=8)
            def _(i0):
                for b in range(8):
                    i = i0 + b
                    pltpu.async_copy(uemb_hbm.at[uidx_s[i]],
                                     ue_out.at[cb + i], sem_u)
                    pltpu.async_copy(iemb_hbm.at[iidx_s[i]],
                                     ie_out.at[cb + i], sem_i)

        pltpu.make_async_copy(uemb_hbm.at[pl.ds(0, _HALF)],
                              ue_out.at[pl.ds(base, _HALF)], sem_u).wait()
        pltpu.make_async_copy(iemb_hbm.at[pl.ds(0, _HALF)],
                              ie_out.at[pl.ds(base, _HALF)], sem_i).wait()

    return k(user_emb, item_emb, user_idx, item_idx)


def _mlp_body(ue_ref, ie_ref, w1u_ref, w1i_ref, b1_ref,
              w2_ref, b2_ref, w3_ref, b3_ref, wo_ref, bo_ref, out_ref):
    ue = ue_ref[...]
    ie = ie_ref[...]
    x = (jnp.dot(ue, w1u_ref[...], preferred_element_type=jnp.float32)
         + jnp.dot(ie, w1i_ref[...], preferred_element_type=jnp.float32)
         + b1_ref[...])
    x = jnp.maximum(x, 0.0)
    x = jnp.dot(x, w2_ref[...], preferred_element_type=jnp.float32) + b2_ref[...]
    x = jnp.maximum(x, 0.0)
    x = jnp.dot(x, w3_ref[...], preferred_element_type=jnp.float32) + b3_ref[...]
    x = jnp.maximum(x, 0.0)
    y = jnp.dot(x, wo_ref[...], preferred_element_type=jnp.float32) + bo_ref[...]
    out_ref[...] = y


_BLK = 1024


def _tc_mlp(ue, ie, W1, b1, W2, b2, W3, b3, Wout, bout):
    w1u = W1[:_D]
    w1i = W1[_D:]
    blk = lambda shape: pl.BlockSpec(shape, lambda i: (i, 0))
    full = lambda shape: pl.BlockSpec(shape, lambda i: (0, 0))
    out = pl.pallas_call(
        _mlp_body,
        grid=(_B // _BLK,),
        in_specs=[
            blk((_BLK, _D)), blk((_BLK, _D)),
            full((_D, 32)), full((_D, 32)), full((1, 32)),
            full((32, 16)), full((1, 16)),
            full((16, 8)), full((1, 8)),
            full((8, 1)), full((1, 1)),
        ],
        out_specs=blk((_BLK, 1)),
        out_shape=jax.ShapeDtypeStruct((_B, 1), jnp.float32),
    )(ue, ie, w1u, w1i, b1[None, :],
      W2, b2[None, :], W3, b3[None, :], Wout, bout[None, :])
    return out[:, 0]


def kernel(user_idx, item_idx, user_emb, item_emb,
           W1, b1, W2, b2, W3, b3, Wout, bout):
    ue, ie = _sc_gather2(user_emb, item_emb, user_idx, item_idx)
    return _tc_mlp(ue, ie, W1, b1, W2, b2, W3, b3, Wout, bout)
